# Initial kernel scaffold; baseline (speedup 1.0000x reference)
import jax
import jax.numpy as jnp
from jax.experimental import pallas as pl


def kernel(x, ids, pe):
    raise NotImplementedError("write your pallas kernel here")


# speedup vs baseline: 1.1946x; 1.1946x over previous
import functools

import jax
import jax.numpy as jnp
from jax import lax
from jax.experimental import pallas as pl
from jax.experimental.pallas import tpu as pltpu
from jax.experimental.pallas import tpu_sc as plsc

B, N, D = 64, 576, 768
R = B * N
NUM_PATCHES = 576

_info = plsc.get_sparse_core_info()
NC, NS, L = _info.num_cores, _info.num_subcores, _info.num_lanes
NW = NC * NS
ROWS_PER_W = R // NW
C = 64
NCHUNK = ROWS_PER_W // C


def _body(x_hbm, ids_hbm, pe_hbm, out_hbm, idx_v, pe_buf, x_buf, sem):
    wid = lax.axis_index("s") * NC + lax.axis_index("c")
    base = wid * ROWS_PER_W

    def chunk(i, carry):
        start = base + i * C
        pltpu.sync_copy(ids_hbm.at[pl.ds(start, C)], idx_v)
        gather = pltpu.async_copy(pe_hbm.at[idx_v], pe_buf, sem)
        pltpu.sync_copy(x_hbm.at[pl.ds(start, C)], x_buf)
        gather.wait()

        def row(r, rc):
            for g in range(D // L):
                sl = pl.ds(g * L, L)
                x_buf[r, sl] = x_buf[r, sl] + pe_buf[r, sl]
            return rc

        lax.fori_loop(0, C, row, 0)
        pltpu.sync_copy(x_buf, out_hbm.at[pl.ds(start, C)])
        return carry

    lax.fori_loop(0, NCHUNK, chunk, 0)


@jax.jit
def kernel(x, ids, pe):
    x2 = x.reshape(R, D)
    ids2 = ids.reshape(R).astype(jnp.int32)
    pe2 = pe.reshape(NUM_PATCHES, D)

    mesh = plsc.VectorSubcoreMesh(core_axis_name="c", subcore_axis_name="s")
    out = pl.kernel(
        _body,
        mesh=mesh,
        out_type=jax.ShapeDtypeStruct((R, D), jnp.float32),
        scratch_types=[
            pltpu.VMEM((C,), jnp.int32),
            pltpu.VMEM((C, D), jnp.float32),
            pltpu.VMEM((C, D), jnp.float32),
            pltpu.SemaphoreType.DMA,
        ],
    )(x2, ids2, pe2)
    return out.reshape(1, B, N, D)


# speedup vs baseline: 1.7323x; 1.4501x over previous
import jax
import jax.numpy as jnp
from jax import lax
from jax.experimental import pallas as pl
from jax.experimental.pallas import tpu as pltpu
from jax.experimental.pallas import tpu_sc as plsc

B, N, D = 64, 576, 768
R = B * N
NUM_PATCHES = 576

_info = plsc.get_sparse_core_info()
NC, NS, L = _info.num_cores, _info.num_subcores, _info.num_lanes
NW = NC * NS
ROWS_PER_W = R // NW
C = 32
NBUF = 2
NCHUNK = ROWS_PER_W // C
NROUND = NCHUNK // NBUF


def _body(x_hbm, ids_hbm, pe_hbm, out_hbm,
          ids_all, xb0, xb1, pb0, pb1, sx0, sx1, sg0, sg1, so0, so1):
    xb = (xb0, xb1)
    pb = (pb0, pb1)
    sx = (sx0, sx1)
    sg = (sg0, sg1)
    so = (so0, so1)

    wid = lax.axis_index("s") * NC + lax.axis_index("c")
    base = wid * ROWS_PER_W
    pltpu.sync_copy(ids_hbm.at[pl.ds(base, ROWS_PER_W)], ids_all)

    def start(b, g, first):
        if not first:
            pltpu.make_async_copy(
                xb[b], out_hbm.at[pl.ds(base, C)], so[b]).wait()
        pltpu.async_copy(x_hbm.at[pl.ds(base + g * C, C)], xb[b], sx[b])
        pltpu.async_copy(
            pe_hbm.at[ids_all.at[pl.ds(g * C, C)]], pb[b], sg[b])

    def finish(b, g):
        pltpu.make_async_copy(
            x_hbm.at[pl.ds(base, C)], xb[b], sx[b]).wait()
        pltpu.make_async_copy(
            pe_hbm.at[ids_all.at[pl.ds(g * C, C)]], pb[b], sg[b]).wait()

        def row(r, rc):
            for gr in range(D // L):
                sl = pl.ds(gr * L, L)
                xb[b][r, sl] = xb[b][r, sl] + pb[b][r, sl]
            return rc

        lax.fori_loop(0, C, row, 0)
        pltpu.async_copy(xb[b], out_hbm.at[pl.ds(base + g * C, C)], so[b])

    start(0, 0, True)
    start(1, 1, True)
    finish(0, 0)
    start(0, 2, False)
    finish(1, 1)

    def round_(k, carry):
        for b in range(NBUF):
            g = k * NBUF + b

            @pl.when(g + 1 < NCHUNK)
            def _():
                start((b + 1) % NBUF, g + 1, False)

            finish(b, g)
        return carry

    lax.fori_loop(1, NROUND, round_, 0)

    for b in range(NBUF):
        pltpu.make_async_copy(xb[b], out_hbm.at[pl.ds(base, C)], so[b]).wait()


@jax.jit
def kernel(x, ids, pe):
    x2 = x.reshape(R, D)
    ids2 = ids.reshape(R).astype(jnp.int32)
    pe2 = pe.reshape(NUM_PATCHES, D)

    mesh = plsc.VectorSubcoreMesh(core_axis_name="c", subcore_axis_name="s")
    out = pl.kernel(
        _body,
        mesh=mesh,
        out_type=jax.ShapeDtypeStruct((R, D), jnp.float32),
        scratch_types=[
            pltpu.VMEM((ROWS_PER_W,), jnp.int32),
            pltpu.VMEM((C, D), jnp.float32),
            pltpu.VMEM((C, D), jnp.float32),
            pltpu.VMEM((C, D), jnp.float32),
            pltpu.VMEM((C, D), jnp.float32),
        ] + [pltpu.SemaphoreType.DMA] * 6,
    )(x2, ids2, pe2)
    return out.reshape(1, B, N, D)
